# baseline (speedup 1.0000x reference)
import functools

import jax
import jax.numpy as jnp
from jax import lax
from jax.experimental import pallas as pl
from jax.experimental.pallas import tpu as pltpu
from jax.experimental.pallas import tpu_sc as plsc

_VOCAB = 100000
_D = 1024
_B = 4
_S = 2048
_NW = 32
_SW = _S // _NW
_CH = 32
_NCH = _SW // _CH

_mesh = plsc.VectorSubcoreMesh(core_axis_name="c", subcore_axis_name="s")


@functools.partial(
    pl.kernel,
    mesh=_mesh,
    out_type=jax.ShapeDtypeStruct((_B * _S, _D), jnp.float32),
    scratch_types=[
        pltpu.VMEM((_SW,), jnp.int32),
        pltpu.VMEM((_SW, _D), jnp.float32),
        pltpu.VMEM((_CH, _D), jnp.float32),
        pltpu.SemaphoreType.DMA,
    ],
)
def _emb_lookup(x_hbm, emb_hbm, pos_hbm, out_hbm, idx_v, pos_v, gbuf, sem):
    cid = lax.axis_index("c")
    sid = lax.axis_index("s")
    wid = sid * 2 + cid
    s0 = wid * _SW

    pltpu.sync_copy(pos_hbm.at[pl.ds(s0, _SW)], pos_v)

    for b in range(_B):
        pltpu.sync_copy(x_hbm.at[pl.ds(b * _S + s0, _SW)], idx_v)
        for h in range(_NCH):
            pltpu.async_copy(
                emb_hbm.at[idx_v.at[pl.ds(h * _CH, _CH)]], gbuf, sem
            ).wait()

            def _row_add(r, _, h=h):
                for j in range(_D // 16):
                    sl = pl.ds(j * 16, 16)
                    gbuf[r, sl] = gbuf[r, sl] + pos_v[h * _CH + r, sl]
                return 0

            lax.fori_loop(0, _CH, _row_add, 0)
            pltpu.sync_copy(
                gbuf, out_hbm.at[pl.ds(b * _S + s0 + h * _CH, _CH)]
            )


def kernel(x, emb, pos_emb):
    xf = x.reshape(_B * _S).astype(jnp.int32)
    out = _emb_lookup(xf, emb, pos_emb)
    return out.reshape(_B, _S, _D)
